# Initial kernel scaffold; baseline (speedup 1.0000x reference)
import jax
import jax.numpy as jnp
from jax.experimental import pallas as pl


def kernel(q, k, v, is_causal, mask, proj_dir):
    raise NotImplementedError("write your pallas kernel here")


# speedup vs baseline: 2.5300x; 2.5300x over previous
import functools
import math

import numpy as np
import jax
import jax.numpy as jnp
from jax import lax
from jax.experimental import pallas as pl
from jax.experimental.pallas import tpu as pltpu
from jax.experimental.pallas import tpu_sc as plsc

_NUM_PROJS = 7
_NBUCKET = 1 << _NUM_PROJS
_BLOCK = 32

_B, _S, _H, _D = 1, 2048, 12, 128
_ROWS = _S * _H
_NB = _S // _BLOCK
_PER_BLK = _ROWS // _NB
_R128 = _ROWS // 128

_BQ = 256
_BK = 256
_NQ = _S // _BQ

_NC, _NS = 2, 16
_NW = _NC * _NS
_RPW = _ROWS // _NW
_CH = 128
_NCHUNK = _RPW // _CH


def _cumsum_rows(x):
    n = x.shape[0]
    k = 1
    while k < n:
        shifted = jnp.concatenate(
            [jnp.zeros((k, x.shape[1]), x.dtype), x[:-k]], axis=0)
        x = x + shifted
        k *= 2
    return x


def _cumsum_lanes(x):
    n = x.shape[1]
    k = 1
    while k < n:
        shifted = jnp.concatenate(
            [jnp.zeros((x.shape[0], k), x.dtype), x[:, :-k]], axis=1)
        x = x + shifted
        k *= 2
    return x


def _hash_col(x, pd, lanes, perm_row, enc_row):
    m = jnp.dot(x, pd, preferred_element_type=jnp.float32)
    bits = (m > 0).astype(jnp.float32)
    binid = jnp.sum(bits * enc_row, axis=1, keepdims=True)
    ohb = (binid == lanes).astype(jnp.float32)
    return jnp.sum(ohb * perm_row, axis=1, keepdims=True)


def _prep_body(q_ref, k_ref, pd_ref, rank_ref, eq_ref):
    lane_i = lax.broadcasted_iota(jnp.int32, (1, _NBUCKET), 1)
    lanes = lane_i.astype(jnp.float32)
    perm_row = (lane_i ^ (lane_i >> 1)).astype(jnp.float32)
    enc_i = lax.broadcasted_iota(jnp.int32, (1, 8), 1)
    enc_row = jnp.where(enc_i < _NUM_PROJS, 1 << enc_i, 0).astype(jnp.float32)
    pd = pd_ref[...]
    qh = _hash_col(q_ref[...], pd, lanes, perm_row, enc_row)
    kh = _hash_col(k_ref[...], pd, lanes, perm_row, enc_row)

    oh = (qh == lanes).astype(jnp.float32)
    counts = jnp.sum(oh, axis=0, keepdims=True)
    cnt_incl = _cumsum_lanes(counts)
    off_excl = cnt_incl - counts
    csum = _cumsum_rows(oh)
    rank = jnp.sum((off_excl + csum - 1.0) * oh, axis=1,
                   keepdims=True)

    r_iota = lax.broadcasted_iota(jnp.int32, (_S, 1), 0).astype(jnp.float32)
    qhs = jnp.sum((cnt_incl <= r_iota).astype(jnp.float32),
                  axis=1, keepdims=True)
    eq = (qhs == kh).astype(jnp.float32)

    rank_ref[...] = jnp.reshape(rank, (1, 1, _S))
    eq_ref[...] = jnp.reshape(eq, (1, 1, _S))


def _prep_call(q2, k2, pd8):
    return pl.pallas_call(
        _prep_body,
        grid=(_H,),
        in_specs=[
            pl.BlockSpec((_S, _D), lambda h: (0, h)),
            pl.BlockSpec((_S, _D), lambda h: (0, h)),
            pl.BlockSpec((_D, 8), lambda h: (0, 0)),
        ],
        out_specs=[
            pl.BlockSpec((1, 1, _S), lambda h: (h, 0, 0)),
            pl.BlockSpec((1, 1, _S), lambda h: (h, 0, 0)),
        ],
        out_shape=[
            jax.ShapeDtypeStruct((_H, 1, _S), jnp.float32),
            jax.ShapeDtypeStruct((_H, 1, _S), jnp.float32),
        ],
    )(q2, k2, pd8)


def _final_body(rank_ref, eq_ref, kmask_ref, dest_ref):
    eq = eq_ref[...]
    e192 = jnp.reshape(eq, (_R128, 128))
    rowsum = jnp.sum(e192, axis=1, keepdims=True)
    ra = lax.broadcasted_iota(jnp.int32, (_NB, _R128), 0)
    ca = lax.broadcasted_iota(jnp.int32, (_NB, _R128), 1)
    seg_a = ((ca >= 3 * ra) & (ca < 3 * ra + 3)).astype(jnp.float32)
    rb = lax.broadcasted_iota(jnp.int32, (_R128, _NB), 0)
    cb = lax.broadcasted_iota(jnp.int32, (_R128, _NB), 1)
    seg_b = ((rb >= 3 * cb) & (rb < 3 * cb + 3)).astype(jnp.float32)
    blksum = jnp.dot(seg_a, rowsum,
                     preferred_element_type=jnp.float32)
    keep = (blksum > 0).astype(jnp.float32)
    km192 = jnp.dot(seg_b, keep,
                    preferred_element_type=jnp.float32)
    km = jnp.broadcast_to(km192, (_R128, 128))
    kmask_ref[...] = jnp.reshape(km, (_H, 1, _S))

    rank = jnp.reshape(rank_ref[...], (_H, _S))
    rk_t = jnp.transpose(rank)
    hoff = lax.broadcasted_iota(
        jnp.int32, (1, _H), 1).astype(jnp.float32) * float(_S)
    dest_ref[...] = (rk_t + hoff).astype(jnp.int32)


def _final_call(rank3, eq3):
    return pl.pallas_call(
        _final_body,
        out_shape=[
            jax.ShapeDtypeStruct((_H, 1, _S), jnp.float32),
            jax.ShapeDtypeStruct((_S, _H), jnp.int32),
        ],
    )(rank3, eq3)


def _sc_scatter(qflat, dest3):
    mesh = plsc.VectorSubcoreMesh(
        core_axis_name="c", subcore_axis_name="s",
        num_cores=_NC, num_subcores=_NS)

    @functools.partial(
        pl.kernel, mesh=mesh,
        out_type=jax.ShapeDtypeStruct((_ROWS, _D), jnp.float32),
        scratch_types=[
            pltpu.VMEM((_NCHUNK, _CH), jnp.int32),
            pltpu.VMEM((_RPW, _D), jnp.float32),
            pltpu.SemaphoreType.DMA,
        ],
    )
    def sc_scatter_kernel(q_hbm, d_hbm, out_hbm, idx_v, rows_v, sem):
        wid = lax.axis_index("s") * _NC + lax.axis_index("c")
        base = wid * _RPW
        pltpu.sync_copy(d_hbm.at[wid], idx_v)
        pltpu.sync_copy(q_hbm.at[pl.ds(base, _RPW)], rows_v)
        for j in range(_NCHUNK):
            pltpu.async_copy(
                rows_v.at[pl.ds(j * _CH, _CH)],
                out_hbm.at[idx_v.at[j]], sem).wait()

    return sc_scatter_kernel(qflat, dest3)


def _sc_gather(aflat, dest3):
    mesh = plsc.VectorSubcoreMesh(
        core_axis_name="c", subcore_axis_name="s",
        num_cores=_NC, num_subcores=_NS)

    @functools.partial(
        pl.kernel, mesh=mesh,
        out_type=jax.ShapeDtypeStruct((_ROWS, _D), jnp.float32),
        scratch_types=[
            pltpu.VMEM((_NCHUNK, _CH), jnp.int32),
            pltpu.VMEM((_RPW, _D), jnp.float32),
            pltpu.SemaphoreType.DMA,
        ],
    )
    def sc_gather_kernel(a_hbm, d_hbm, out_hbm, idx_v, rows_v, sem):
        wid = lax.axis_index("s") * _NC + lax.axis_index("c")
        base = wid * _RPW
        pltpu.sync_copy(d_hbm.at[wid], idx_v)
        for j in range(_NCHUNK):
            pltpu.async_copy(
                a_hbm.at[idx_v.at[j]],
                rows_v.at[pl.ds(j * _CH, _CH)], sem).wait()
        pltpu.sync_copy(rows_v, out_hbm.at[pl.ds(base, _RPW)])

    return sc_gather_kernel(aflat, dest3)


def _attn_body(q_ref, k_ref, v_ref, m_ref, o_ref):
    qi = pl.program_id(1)
    scale = 1.0 / math.sqrt(float(_D))
    q = q_ref[...] * scale
    rows = lax.broadcasted_iota(jnp.int32, (_BQ, _BK), 0)
    cols = lax.broadcasted_iota(jnp.int32, (_BQ, _BK), 1)
    tri = rows >= cols

    def step(kj, carry):
        m, l, acc = carry
        kb = k_ref[pl.ds(kj * _BK, _BK), :]
        vb = v_ref[pl.ds(kj * _BK, _BK), :]
        s = lax.dot_general(q, kb, (((1,), (1,)), ((), ())),
                            preferred_element_type=jnp.float32)
        km = m_ref[0, :, pl.ds(kj * _BK, _BK)]
        s = s * km
        s = lax.cond(
            kj == qi,
            lambda x: jnp.where(tri, x, -jnp.inf),
            lambda x: x, s)
        m_new = jnp.maximum(m, jnp.max(s, axis=1, keepdims=True))
        p = jnp.exp(s - m_new)
        alpha = jnp.exp(m - m_new)
        l_new = l * alpha + jnp.sum(p, axis=1, keepdims=True)
        acc_new = acc * alpha + lax.dot_general(
            p, vb, (((1,), (0,)), ((), ())),
            preferred_element_type=jnp.float32)
        return m_new, l_new, acc_new

    m0 = jnp.full((_BQ, 1), -jnp.inf, jnp.float32)
    l0 = jnp.zeros((_BQ, 1), jnp.float32)
    a0 = jnp.zeros((_BQ, _D), jnp.float32)
    m, l, acc = lax.fori_loop(0, qi + 1, step, (m0, l0, a0))
    o_ref[...] = acc / l


def _attn_call(qs, k2, v2, kmask3):
    return pl.pallas_call(
        _attn_body,
        grid=(_H, _NQ),
        in_specs=[
            pl.BlockSpec((_BQ, _D), lambda h, qi: (h * _NQ + qi, 0)),
            pl.BlockSpec((_S, _D), lambda h, qi: (0, h)),
            pl.BlockSpec((_S, _D), lambda h, qi: (0, h)),
            pl.BlockSpec((1, 1, _S), lambda h, qi: (h, 0, 0)),
        ],
        out_specs=pl.BlockSpec((_BQ, _D), lambda h, qi: (h * _NQ + qi, 0)),
        out_shape=jax.ShapeDtypeStruct((_ROWS, _D), jnp.float32),
    )(qs, k2, v2, kmask3)


def kernel(q, k, v, is_causal, mask, proj_dir):
    B, S, H, D = q.shape
    q2 = q.reshape(S, H * D)
    k2 = k.reshape(S, H * D)
    v2 = v.reshape(S, H * D)
    pd8 = jnp.pad(proj_dir[0, 0], ((0, 0), (0, 1)))

    rank3, eq3 = _prep_call(q2, k2, pd8)
    kmask3, dest2d = _final_call(rank3, eq3)
    dest3 = dest2d.reshape(_NW, _NCHUNK, _CH)

    qflat = q.reshape(S * H, D)
    qs = _sc_scatter(qflat, dest3)
    oa = _attn_call(qs, k2, v2, kmask3)
    out = _sc_gather(oa, dest3)
    return out.reshape(B, S, H, D)
